# baseline (speedup 1.0000x reference)
import functools

import jax
import jax.numpy as jnp
from jax import lax
from jax.experimental import pallas as pl
from jax.experimental.pallas import tpu as pltpu
from jax.experimental.pallas import tpu_sc as plsc

N = 10000
E = 320000
D = 128
H = 128
C = 40

NC = 2
NS = 16
K = 128
E_PAD = 327680
TRASH_DST = 10200
EROWS = E_PAD // K
RPT_DEG = EROWS // (NC * NS)
RPT_AGG = EROWS // NS
MB = 8
ACC_N = 10240
NPT = ACC_N // NS
ZROWS = 128
QN = 2560
QACC = 2688
QPT = QACC // NS
SROWS = 624


def _fill_zeros(ref, nrows, ncols):
    def body(i, carry):
        for j in range(ncols // 16):
            ref[i, pl.ds(j * 16, 16)] = jnp.zeros((16,), jnp.float32)
        return carry
    lax.fori_loop(0, nrows, body, 0)


@functools.cache
def _sc_kernels():
    mesh = plsc.VectorSubcoreMesh(core_axis_name="c", subcore_axis_name="s",
                                  num_cores=NC, num_subcores=NS)

    degree_kernel = functools.partial(
        pl.kernel,
        out_type=jax.ShapeDtypeStruct((NC, ACC_N, 16), jnp.float32),
        mesh=mesh,
        scratch_types=[
            pltpu.VMEM((RPT_DEG, K), jnp.int32),
            pltpu.VMEM((K, 16), jnp.float32),
            pltpu.VMEM((ZROWS, 16), jnp.float32),
            pltpu.VMEM_SHARED((ACC_N, 16), jnp.float32),
        ],
    )(_degree_body)

    agg_scratch = [
        pltpu.VMEM((MB, K), jnp.int32),
        pltpu.VMEM((MB, K), jnp.int32),
        pltpu.VMEM((MB, K), jnp.int32),
        pltpu.VMEM((ZROWS, H), jnp.float32),
        pltpu.VMEM_SHARED((ACC_N, H), jnp.float32),
        pltpu.VMEM_SHARED((QACC, H), jnp.float32),
        pltpu.SemaphoreType.DMA,
    ]
    agg_kernels = [
        functools.partial(
            pl.kernel,
            out_type=jax.ShapeDtypeStruct((NC, QACC, H), jnp.float32),
            mesh=mesh,
            scratch_types=list(agg_scratch),
        )(functools.partial(_agg_body, 2 * call))
        for call in range(2)
    ]

    return degree_kernel, agg_kernels


def _degree_body(dst_hbm, out_hbm, dst_v, ones_v, zbuf_v, acc_sh):
    c = lax.axis_index("c")
    s = lax.axis_index("s")
    row0 = (c * NS + s) * RPT_DEG
    pltpu.sync_copy(dst_hbm.at[pl.ds(row0, RPT_DEG)], dst_v)

    def fill_ones(i, carry):
        ones_v[i, pl.ds(0, 16)] = jnp.ones((16,), jnp.float32)
        return carry
    lax.fori_loop(0, K, fill_ones, 0)
    _fill_zeros(zbuf_v, ZROWS, 16)

    def zcp(i, carry):
        pltpu.sync_copy(zbuf_v, acc_sh.at[pl.ds(s * NPT + i * ZROWS, ZROWS)])
        return carry
    lax.fori_loop(0, NPT // ZROWS, zcp, 0)
    plsc.subcore_barrier()

    def body(j, carry):
        pltpu.sync_copy(ones_v, acc_sh.at[dst_v.at[j]], add=True)
        return carry
    lax.fori_loop(0, RPT_DEG, body, 0)
    plsc.subcore_barrier()

    def ocp(i, carry):
        off = s * NPT + i * ZROWS
        pltpu.sync_copy(acc_sh.at[pl.ds(off, ZROWS)], zbuf_v)
        pltpu.sync_copy(zbuf_v, out_hbm.at[c, pl.ds(off, ZROWS)])
        return carry
    lax.fori_loop(0, NPT // ZROWS, ocp, 0)


def _agg_body(qbase2, y_hbm, src_hbm, dst_hbm, out_hbm,
              src_v, dst_v, dstr_v, msg_v, y_sh, acc_sh, sem):
    c = lax.axis_index("c")
    s = lax.axis_index("s")
    qbase = (qbase2 + c) * QN

    for t, sz in ((0, 128), (128, 128), (256, 128), (384, 128), (512, 112)):
        pltpu.sync_copy(y_hbm.at[pl.ds(s * SROWS + t, sz)],
                        msg_v.at[pl.ds(0, sz)])
        pltpu.sync_copy(msg_v.at[pl.ds(0, sz)],
                        y_sh.at[pl.ds(s * SROWS + t, sz)])

    @pl.when(s == 0)
    def _():
        pltpu.sync_copy(y_hbm.at[pl.ds(NS * SROWS, N - NS * SROWS)],
                        msg_v.at[pl.ds(0, N - NS * SROWS)])
        pltpu.sync_copy(msg_v.at[pl.ds(0, N - NS * SROWS)],
                        y_sh.at[pl.ds(NS * SROWS, N - NS * SROWS)])

    _fill_zeros(msg_v, ZROWS, H)
    pltpu.sync_copy(msg_v, acc_sh.at[pl.ds(s * QPT, ZROWS)])
    pltpu.sync_copy(msg_v.at[pl.ds(0, QPT - ZROWS)],
                    acc_sh.at[pl.ds(s * QPT + ZROWS, QPT - ZROWS)])
    plsc.subcore_barrier()

    row0 = s * RPT_AGG

    def blk(m, carry):
        pltpu.sync_copy(src_hbm.at[pl.ds(row0 + m * MB, MB)], src_v)
        pltpu.sync_copy(dst_hbm.at[pl.ds(row0 + m * MB, MB)], dst_v)

        def chunk(jj, carry2):
            descs = []
            for g in range(K // 16):
                sl = pl.ds(g * 16, 16)
                v16 = src_v[jj, sl]
                descs += [
                    pltpu.async_copy(y_sh.at[pl.ds(v16[t], 1)],
                                     msg_v.at[pl.ds(g * 16 + t, 1)], sem)
                    for t in range(16)
                ]
                local = dst_v[jj, sl] - qbase
                ok = (local >= 0) & (local < QN)
                dstr_v[jj, sl] = jnp.where(ok, local, QN)
            for d in descs:
                d.wait()
            pltpu.sync_copy(msg_v, acc_sh.at[dstr_v.at[jj]], add=True)
            return carry2
        lax.fori_loop(0, MB, chunk, 0)
        return carry
    lax.fori_loop(0, RPT_AGG // MB, blk, 0)
    plsc.subcore_barrier()

    pltpu.sync_copy(acc_sh.at[pl.ds(s * QPT, ZROWS)], msg_v)
    pltpu.sync_copy(msg_v, out_hbm.at[c, pl.ds(s * QPT, ZROWS)])
    pltpu.sync_copy(acc_sh.at[pl.ds(s * QPT + ZROWS, QPT - ZROWS)],
                    msg_v.at[pl.ds(0, QPT - ZROWS)])
    pltpu.sync_copy(msg_v.at[pl.ds(0, QPT - ZROWS)],
                    out_hbm.at[c, pl.ds(s * QPT + ZROWS, QPT - ZROWS)])


BN = 80


def _k1_body(dega_ref, x_ref, w_ref, dinv_ref, y_ref):
    dega = dega_ref[...]
    deg = 1.0 + dega[0, :, 0] + dega[1, :, 0]
    dinv = lax.rsqrt(deg)
    dinv_ref[...] = dinv[:, None]
    xw = jnp.dot(x_ref[...], w_ref[...], preferred_element_type=jnp.float32)
    y_ref[...] = xw * dinv[:, None]


def _acc_select(accA_ref, accB_ref):
    qn = pl.program_id(0) // (QN // BN)
    a = accA_ref[...][0]
    b = accB_ref[...][0]
    return jnp.where(qn < 2, a, b)


def _k_mid_body(accA_ref, accB_ref, y_ref, dinv_ref, b_ref, w_ref, out_ref):
    sacc = _acc_select(accA_ref, accB_ref) + y_ref[...]
    dinv = dinv_ref[...]
    h = jnp.maximum(dinv * sacc + b_ref[...], 0.0)
    out_ref[...] = jnp.dot(h, w_ref[...], preferred_element_type=jnp.float32) * dinv


def _k_last_body(accA_ref, accB_ref, y_ref, dinv_ref, b_ref, wc_ref, bc_ref,
                 out_ref, h_ref):
    sacc = _acc_select(accA_ref, accB_ref) + y_ref[...]
    h = jnp.maximum(dinv_ref[...] * sacc + b_ref[...], 0.0)
    h_ref[...] = h
    out_ref[...] = jnp.dot(h, wc_ref[...], preferred_element_type=jnp.float32) + bc_ref[...]


def _acc_spec(lo):
    def imap(i):
        qn = i // (QN // BN)
        cc = jnp.clip(qn - lo, 0, 1)
        return (cc, i % (QN // BN), 0)
    return pl.BlockSpec((1, BN, H), imap)


_full2 = lambda r, c: pl.BlockSpec((r, c), lambda i: (0, 0))
_rows2 = lambda c: pl.BlockSpec((BN, c), lambda i: (i, 0))
_rows3 = lambda c: pl.BlockSpec((NC, BN, c), lambda i: (0, i, 0))

_k1_call = pl.pallas_call(
    _k1_body,
    grid=(N // BN,),
    in_specs=[_rows3(16), _rows2(D), _full2(D, H)],
    out_specs=[_rows2(1), _rows2(H)],
    out_shape=[jax.ShapeDtypeStruct((N, 1), jnp.float32),
               jax.ShapeDtypeStruct((N, H), jnp.float32)],
)

_k_mid_call = pl.pallas_call(
    _k_mid_body,
    grid=(N // BN,),
    in_specs=[_acc_spec(0), _acc_spec(2), _rows2(H), _rows2(1),
              _full2(1, H), _full2(H, H)],
    out_specs=_rows2(H),
    out_shape=jax.ShapeDtypeStruct((N, H), jnp.float32),
)

_k_last_call = pl.pallas_call(
    _k_last_body,
    grid=(N // BN,),
    in_specs=[_acc_spec(0), _acc_spec(2), _rows2(H), _rows2(1),
              _full2(1, H), _full2(H, C), _full2(1, C)],
    out_specs=[_rows2(C), _rows2(H)],
    out_shape=[jax.ShapeDtypeStruct((N, C), jnp.float32),
               jax.ShapeDtypeStruct((N, H), jnp.float32)],
)


def kernel(x, edge_index, W1, b1, W2, b2, W3, b3, Wc, bc):
    npad = E_PAD - E
    srcf = jnp.concatenate(
        [edge_index[0].astype(jnp.int32), jnp.zeros((npad,), jnp.int32)])
    dstf = jnp.concatenate(
        [edge_index[1].astype(jnp.int32),
         jnp.full((npad,), TRASH_DST, jnp.int32)])
    src2 = srcf.reshape(EROWS, K)
    dst2 = dstf.reshape(EROWS, K)

    _degree_kernel, _agg_kernels = _sc_kernels()
    dega = _degree_kernel(dst2)

    def layer_agg(y):
        accA = _agg_kernels[0](y, src2, dst2)
        accB = _agg_kernels[1](y, src2, dst2)
        return accA, accB

    dinv, y1 = _k1_call(dega, x, W1)
    accA, accB = layer_agg(y1)
    y2 = _k_mid_call(accA, accB, y1, dinv, b1.reshape(1, H), W2)
    accA, accB = layer_agg(y2)
    y3 = _k_mid_call(accA, accB, y2, dinv, b2.reshape(1, H), W3)
    accA, accB = layer_agg(y3)
    out, h = _k_last_call(accA, accB, y3, dinv, b3.reshape(1, H), Wc,
                          bc.reshape(1, C))
    return (out, h)


# speedup vs baseline: 1.1225x; 1.1225x over previous
import functools

import jax
import jax.numpy as jnp
from jax import lax
from jax.experimental import pallas as pl
from jax.experimental.pallas import tpu as pltpu
from jax.experimental.pallas import tpu_sc as plsc

N = 10000
E = 320000
D = 128
H = 128
C = 40

NC = 2
NS = 16
K = 128
E_PAD = 327680
TRASH_DST = 10200
EROWS = E_PAD // K
RPT_DEG = EROWS // (NC * NS)
RPT_AGG = EROWS // NS
MB = 8
ACC_N = 10240
NPT = ACC_N // NS
ZROWS = 128
QN = 2560
QACC = 2688
QPT = QACC // NS
SROWS = 624


def _fill_zeros(ref, nrows, ncols):
    def body(i, carry):
        for j in range(ncols // 16):
            ref[i, pl.ds(j * 16, 16)] = jnp.zeros((16,), jnp.float32)
        return carry
    lax.fori_loop(0, nrows, body, 0)


@functools.cache
def _sc_kernels():
    mesh = plsc.VectorSubcoreMesh(core_axis_name="c", subcore_axis_name="s",
                                  num_cores=NC, num_subcores=NS)

    degree_kernel = functools.partial(
        pl.kernel,
        out_type=jax.ShapeDtypeStruct((NC, ACC_N, 16), jnp.float32),
        mesh=mesh,
        scratch_types=[
            pltpu.VMEM((RPT_DEG, K), jnp.int32),
            pltpu.VMEM((K, 16), jnp.float32),
            pltpu.VMEM((ZROWS, 16), jnp.float32),
            pltpu.VMEM_SHARED((ACC_N, 16), jnp.float32),
        ],
    )(_degree_body)

    agg_scratch = [
        pltpu.VMEM((MB, K), jnp.int32),
        pltpu.VMEM((MB, K), jnp.int32),
        pltpu.VMEM((MB, K), jnp.int32),
        pltpu.VMEM((ZROWS, H), jnp.float32),
        pltpu.VMEM_SHARED((ACC_N, H), jnp.float32),
        pltpu.VMEM_SHARED((QACC, H), jnp.float32),
        pltpu.SemaphoreType.DMA,
    ]
    agg_kernels = [
        functools.partial(
            pl.kernel,
            out_type=jax.ShapeDtypeStruct((NC, QACC, H), jnp.float32),
            mesh=mesh,
            scratch_types=list(agg_scratch),
        )(functools.partial(_agg_body, 2 * call))
        for call in range(2)
    ]

    return degree_kernel, agg_kernels


def _degree_body(dst_hbm, out_hbm, dst_v, ones_v, zbuf_v, acc_sh):
    c = lax.axis_index("c")
    s = lax.axis_index("s")
    row0 = (c * NS + s) * RPT_DEG
    pltpu.sync_copy(dst_hbm.at[pl.ds(row0, RPT_DEG)], dst_v)

    def fill_ones(i, carry):
        ones_v[i, pl.ds(0, 16)] = jnp.ones((16,), jnp.float32)
        return carry
    lax.fori_loop(0, K, fill_ones, 0)
    _fill_zeros(zbuf_v, ZROWS, 16)

    def zcp(i, carry):
        pltpu.sync_copy(zbuf_v, acc_sh.at[pl.ds(s * NPT + i * ZROWS, ZROWS)])
        return carry
    lax.fori_loop(0, NPT // ZROWS, zcp, 0)
    plsc.subcore_barrier()

    def body(j, carry):
        pltpu.sync_copy(ones_v, acc_sh.at[dst_v.at[j]], add=True)
        return carry
    lax.fori_loop(0, RPT_DEG, body, 0)
    plsc.subcore_barrier()

    def ocp(i, carry):
        off = s * NPT + i * ZROWS
        pltpu.sync_copy(acc_sh.at[pl.ds(off, ZROWS)], zbuf_v)
        pltpu.sync_copy(zbuf_v, out_hbm.at[c, pl.ds(off, ZROWS)])
        return carry
    lax.fori_loop(0, NPT // ZROWS, ocp, 0)


def _agg_body(qbase2, y_hbm, src_hbm, dst_hbm, out_hbm,
              src_v, dst_v, dstr_v, msg_v, y_sh, acc_sh, sem):
    c = lax.axis_index("c")
    s = lax.axis_index("s")
    qbase = (qbase2 + c) * QN

    for t, sz in ((0, 128), (128, 128), (256, 128), (384, 128), (512, 112)):
        pltpu.sync_copy(y_hbm.at[pl.ds(s * SROWS + t, sz)],
                        msg_v.at[pl.ds(0, sz)])
        pltpu.sync_copy(msg_v.at[pl.ds(0, sz)],
                        y_sh.at[pl.ds(s * SROWS + t, sz)])

    @pl.when(s == 0)
    def _():
        pltpu.sync_copy(y_hbm.at[pl.ds(NS * SROWS, N - NS * SROWS)],
                        msg_v.at[pl.ds(0, N - NS * SROWS)])
        pltpu.sync_copy(msg_v.at[pl.ds(0, N - NS * SROWS)],
                        y_sh.at[pl.ds(NS * SROWS, N - NS * SROWS)])

    _fill_zeros(msg_v, ZROWS, H)
    pltpu.sync_copy(msg_v, acc_sh.at[pl.ds(s * QPT, ZROWS)])
    pltpu.sync_copy(msg_v.at[pl.ds(0, QPT - ZROWS)],
                    acc_sh.at[pl.ds(s * QPT + ZROWS, QPT - ZROWS)])
    plsc.subcore_barrier()

    row0 = s * RPT_AGG

    def blk(m, carry):
        pltpu.sync_copy(src_hbm.at[pl.ds(row0 + m * MB, MB)], src_v)
        pltpu.sync_copy(dst_hbm.at[pl.ds(row0 + m * MB, MB)], dst_v)

        def chunk(jj, carry2):
            prev = None
            for g in range(K // 16):
                sl = pl.ds(g * 16, 16)
                v16 = src_v[jj, sl]
                descs = [
                    pltpu.async_copy(y_sh.at[pl.ds(v16[t], 1)],
                                     msg_v.at[pl.ds(g * 16 + t, 1)], sem)
                    for t in range(16)
                ]
                local = dst_v[jj, sl] - qbase
                ok = (local >= 0) & (local < QN)
                dstr_v[jj, sl] = jnp.where(ok, local, QN)
                if prev is not None:
                    for d in prev:
                        d.wait()
                prev = descs
            for d in prev:
                d.wait()
            pltpu.sync_copy(msg_v, acc_sh.at[dstr_v.at[jj]], add=True)
            return carry2
        lax.fori_loop(0, MB, chunk, 0)
        return carry
    lax.fori_loop(0, RPT_AGG // MB, blk, 0)
    plsc.subcore_barrier()

    pltpu.sync_copy(acc_sh.at[pl.ds(s * QPT, ZROWS)], msg_v)
    pltpu.sync_copy(msg_v, out_hbm.at[c, pl.ds(s * QPT, ZROWS)])
    pltpu.sync_copy(acc_sh.at[pl.ds(s * QPT + ZROWS, QPT - ZROWS)],
                    msg_v.at[pl.ds(0, QPT - ZROWS)])
    pltpu.sync_copy(msg_v.at[pl.ds(0, QPT - ZROWS)],
                    out_hbm.at[c, pl.ds(s * QPT + ZROWS, QPT - ZROWS)])


BN = 80


def _k1_body(dega_ref, x_ref, w_ref, dinv_ref, y_ref):
    dega = dega_ref[...]
    deg = 1.0 + dega[0, :, 0] + dega[1, :, 0]
    dinv = lax.rsqrt(deg)
    dinv_ref[...] = dinv[:, None]
    xw = jnp.dot(x_ref[...], w_ref[...], preferred_element_type=jnp.float32)
    y_ref[...] = xw * dinv[:, None]


def _acc_select(accA_ref, accB_ref):
    qn = pl.program_id(0) // (QN // BN)
    a = accA_ref[...][0]
    b = accB_ref[...][0]
    return jnp.where(qn < 2, a, b)


def _k_mid_body(accA_ref, accB_ref, y_ref, dinv_ref, b_ref, w_ref, out_ref):
    sacc = _acc_select(accA_ref, accB_ref) + y_ref[...]
    dinv = dinv_ref[...]
    h = jnp.maximum(dinv * sacc + b_ref[...], 0.0)
    out_ref[...] = jnp.dot(h, w_ref[...], preferred_element_type=jnp.float32) * dinv


def _k_last_body(accA_ref, accB_ref, y_ref, dinv_ref, b_ref, wc_ref, bc_ref,
                 out_ref, h_ref):
    sacc = _acc_select(accA_ref, accB_ref) + y_ref[...]
    h = jnp.maximum(dinv_ref[...] * sacc + b_ref[...], 0.0)
    h_ref[...] = h
    out_ref[...] = jnp.dot(h, wc_ref[...], preferred_element_type=jnp.float32) + bc_ref[...]


def _acc_spec(lo):
    def imap(i):
        qn = i // (QN // BN)
        cc = jnp.clip(qn - lo, 0, 1)
        return (cc, i % (QN // BN), 0)
    return pl.BlockSpec((1, BN, H), imap)


_full2 = lambda r, c: pl.BlockSpec((r, c), lambda i: (0, 0))
_rows2 = lambda c: pl.BlockSpec((BN, c), lambda i: (i, 0))
_rows3 = lambda c: pl.BlockSpec((NC, BN, c), lambda i: (0, i, 0))

_k1_call = pl.pallas_call(
    _k1_body,
    grid=(N // BN,),
    in_specs=[_rows3(16), _rows2(D), _full2(D, H)],
    out_specs=[_rows2(1), _rows2(H)],
    out_shape=[jax.ShapeDtypeStruct((N, 1), jnp.float32),
               jax.ShapeDtypeStruct((N, H), jnp.float32)],
)

_k_mid_call = pl.pallas_call(
    _k_mid_body,
    grid=(N // BN,),
    in_specs=[_acc_spec(0), _acc_spec(2), _rows2(H), _rows2(1),
              _full2(1, H), _full2(H, H)],
    out_specs=_rows2(H),
    out_shape=jax.ShapeDtypeStruct((N, H), jnp.float32),
)

_k_last_call = pl.pallas_call(
    _k_last_body,
    grid=(N // BN,),
    in_specs=[_acc_spec(0), _acc_spec(2), _rows2(H), _rows2(1),
              _full2(1, H), _full2(H, C), _full2(1, C)],
    out_specs=[_rows2(C), _rows2(H)],
    out_shape=[jax.ShapeDtypeStruct((N, C), jnp.float32),
               jax.ShapeDtypeStruct((N, H), jnp.float32)],
)


def kernel(x, edge_index, W1, b1, W2, b2, W3, b3, Wc, bc):
    npad = E_PAD - E
    srcf = jnp.concatenate(
        [edge_index[0].astype(jnp.int32), jnp.zeros((npad,), jnp.int32)])
    dstf = jnp.concatenate(
        [edge_index[1].astype(jnp.int32),
         jnp.full((npad,), TRASH_DST, jnp.int32)])
    src2 = srcf.reshape(EROWS, K)
    dst2 = dstf.reshape(EROWS, K)

    _degree_kernel, _agg_kernels = _sc_kernels()
    dega = _degree_kernel(dst2)

    def layer_agg(y):
        accA = _agg_kernels[0](y, src2, dst2)
        accB = _agg_kernels[1](y, src2, dst2)
        return accA, accB

    dinv, y1 = _k1_call(dega, x, W1)
    accA, accB = layer_agg(y1)
    y2 = _k_mid_call(accA, accB, y1, dinv, b1.reshape(1, H), W2)
    accA, accB = layer_agg(y2)
    y3 = _k_mid_call(accA, accB, y2, dinv, b2.reshape(1, H), W3)
    accA, accB = layer_agg(y3)
    out, h = _k_last_call(accA, accB, y3, dinv, b3.reshape(1, H), Wc,
                          bc.reshape(1, C))
    return (out, h)


# speedup vs baseline: 1.1233x; 1.0007x over previous
import functools

import jax
import jax.numpy as jnp
from jax import lax
from jax.experimental import pallas as pl
from jax.experimental.pallas import tpu as pltpu
from jax.experimental.pallas import tpu_sc as plsc

N = 10000
E = 320000
D = 128
H = 128
C = 40

NC = 2
NS = 16
K = 128
E_PAD = 327680
TRASH_DST = 10200
EROWS = E_PAD // K
RPT_DEG = EROWS // (NC * NS)
RPT_AGG = EROWS // NS
MB = 8
ACC_N = 10240
NPT = ACC_N // NS
ZROWS = 128
QN = 2560
QACC = 2688
QPT = QACC // NS
SROWS = 624


def _fill_zeros(ref, nrows, ncols):
    def body(i, carry):
        for j in range(ncols // 16):
            ref[i, pl.ds(j * 16, 16)] = jnp.zeros((16,), jnp.float32)
        return carry
    lax.fori_loop(0, nrows, body, 0)


@functools.cache
def _sc_kernels():
    mesh = plsc.VectorSubcoreMesh(core_axis_name="c", subcore_axis_name="s",
                                  num_cores=NC, num_subcores=NS)

    degree_kernel = functools.partial(
        pl.kernel,
        out_type=jax.ShapeDtypeStruct((NC, ACC_N, 16), jnp.float32),
        mesh=mesh,
        scratch_types=[
            pltpu.VMEM((RPT_DEG, K), jnp.int32),
            pltpu.VMEM((K, 16), jnp.float32),
            pltpu.VMEM((ZROWS, 16), jnp.float32),
            pltpu.VMEM_SHARED((ACC_N, 16), jnp.float32),
        ],
    )(_degree_body)

    agg_scratch = [
        pltpu.VMEM((MB, K), jnp.int32),
        pltpu.VMEM((MB, K), jnp.int32),
        pltpu.VMEM((MB, K), jnp.int32),
        pltpu.VMEM((ZROWS, H), jnp.float32),
        pltpu.VMEM_SHARED((ACC_N, H), jnp.float32),
        pltpu.VMEM_SHARED((QACC, H), jnp.float32),
        pltpu.SemaphoreType.DMA,
        pltpu.SemaphoreType.DMA,
    ]
    agg_kernels = [
        functools.partial(
            pl.kernel,
            out_type=jax.ShapeDtypeStruct((NC, QACC, H), jnp.float32),
            mesh=mesh,
            scratch_types=list(agg_scratch),
        )(functools.partial(_agg_body, 2 * call))
        for call in range(2)
    ]

    return degree_kernel, agg_kernels


def _degree_body(dst_hbm, out_hbm, dst_v, ones_v, zbuf_v, acc_sh):
    c = lax.axis_index("c")
    s = lax.axis_index("s")
    row0 = (c * NS + s) * RPT_DEG
    pltpu.sync_copy(dst_hbm.at[pl.ds(row0, RPT_DEG)], dst_v)

    def fill_ones(i, carry):
        ones_v[i, pl.ds(0, 16)] = jnp.ones((16,), jnp.float32)
        return carry
    lax.fori_loop(0, K, fill_ones, 0)
    _fill_zeros(zbuf_v, ZROWS, 16)

    def zcp(i, carry):
        pltpu.sync_copy(zbuf_v, acc_sh.at[pl.ds(s * NPT + i * ZROWS, ZROWS)])
        return carry
    lax.fori_loop(0, NPT // ZROWS, zcp, 0)
    plsc.subcore_barrier()

    def body(j, carry):
        pltpu.sync_copy(ones_v, acc_sh.at[dst_v.at[j]], add=True)
        return carry
    lax.fori_loop(0, RPT_DEG, body, 0)
    plsc.subcore_barrier()

    def ocp(i, carry):
        off = s * NPT + i * ZROWS
        pltpu.sync_copy(acc_sh.at[pl.ds(off, ZROWS)], zbuf_v)
        pltpu.sync_copy(zbuf_v, out_hbm.at[c, pl.ds(off, ZROWS)])
        return carry
    lax.fori_loop(0, NPT // ZROWS, ocp, 0)


def _agg_body(qbase2, y_hbm, src_hbm, dst_hbm, out_hbm,
              src_v, dst_v, dstr_v, msg_v, y_sh, acc_sh, sem, sem2):
    c = lax.axis_index("c")
    s = lax.axis_index("s")
    qbase = (qbase2 + c) * QN

    for t, sz in ((0, 128), (128, 128), (256, 128), (384, 128), (512, 112)):
        pltpu.sync_copy(y_hbm.at[pl.ds(s * SROWS + t, sz)],
                        msg_v.at[pl.ds(0, sz)])
        pltpu.sync_copy(msg_v.at[pl.ds(0, sz)],
                        y_sh.at[pl.ds(s * SROWS + t, sz)])

    @pl.when(s == 0)
    def _():
        pltpu.sync_copy(y_hbm.at[pl.ds(NS * SROWS, N - NS * SROWS)],
                        msg_v.at[pl.ds(0, N - NS * SROWS)])
        pltpu.sync_copy(msg_v.at[pl.ds(0, N - NS * SROWS)],
                        y_sh.at[pl.ds(NS * SROWS, N - NS * SROWS)])

    _fill_zeros(msg_v, ZROWS, H)
    pltpu.sync_copy(msg_v, acc_sh.at[pl.ds(s * QPT, ZROWS)])
    pltpu.sync_copy(msg_v.at[pl.ds(0, QPT - ZROWS)],
                    acc_sh.at[pl.ds(s * QPT + ZROWS, QPT - ZROWS)])
    plsc.subcore_barrier()

    row0 = s * RPT_AGG

    def blk(m, carry):
        pltpu.sync_copy(src_hbm.at[pl.ds(row0 + m * MB, MB)], src_v)
        pltpu.sync_copy(dst_hbm.at[pl.ds(row0 + m * MB, MB)], dst_v)

        def chunk(jj, carry2):
            prev = None
            sems = (sem, sem2)
            for g in range(K // 16):
                sl = pl.ds(g * 16, 16)
                v16 = src_v[jj, sl]
                descs = [
                    pltpu.async_copy(y_sh.at[pl.ds(v16[t], 1)],
                                     msg_v.at[pl.ds(g * 16 + t, 1)],
                                     sems[g % 2])
                    for t in range(16)
                ]
                local = dst_v[jj, sl] - qbase
                ok = (local >= 0) & (local < QN)
                dstr_v[jj, sl] = jnp.where(ok, local, QN)
                if prev is not None:
                    for d in prev:
                        d.wait()
                prev = descs
            for d in prev:
                d.wait()
            pltpu.sync_copy(msg_v, acc_sh.at[dstr_v.at[jj]], add=True)
            return carry2
        lax.fori_loop(0, MB, chunk, 0)
        return carry
    lax.fori_loop(0, RPT_AGG // MB, blk, 0)
    plsc.subcore_barrier()

    pltpu.sync_copy(acc_sh.at[pl.ds(s * QPT, ZROWS)], msg_v)
    pltpu.sync_copy(msg_v, out_hbm.at[c, pl.ds(s * QPT, ZROWS)])
    pltpu.sync_copy(acc_sh.at[pl.ds(s * QPT + ZROWS, QPT - ZROWS)],
                    msg_v.at[pl.ds(0, QPT - ZROWS)])
    pltpu.sync_copy(msg_v.at[pl.ds(0, QPT - ZROWS)],
                    out_hbm.at[c, pl.ds(s * QPT + ZROWS, QPT - ZROWS)])


BN = 80


def _k1_body(dega_ref, x_ref, w_ref, dinv_ref, y_ref):
    dega = dega_ref[...]
    deg = 1.0 + dega[0, :, 0] + dega[1, :, 0]
    dinv = lax.rsqrt(deg)
    dinv_ref[...] = dinv[:, None]
    xw = jnp.dot(x_ref[...], w_ref[...], preferred_element_type=jnp.float32)
    y_ref[...] = xw * dinv[:, None]


def _acc_select(accA_ref, accB_ref):
    qn = pl.program_id(0) // (QN // BN)
    a = accA_ref[...][0]
    b = accB_ref[...][0]
    return jnp.where(qn < 2, a, b)


def _k_mid_body(accA_ref, accB_ref, y_ref, dinv_ref, b_ref, w_ref, out_ref):
    sacc = _acc_select(accA_ref, accB_ref) + y_ref[...]
    dinv = dinv_ref[...]
    h = jnp.maximum(dinv * sacc + b_ref[...], 0.0)
    out_ref[...] = jnp.dot(h, w_ref[...], preferred_element_type=jnp.float32) * dinv


def _k_last_body(accA_ref, accB_ref, y_ref, dinv_ref, b_ref, wc_ref, bc_ref,
                 out_ref, h_ref):
    sacc = _acc_select(accA_ref, accB_ref) + y_ref[...]
    h = jnp.maximum(dinv_ref[...] * sacc + b_ref[...], 0.0)
    h_ref[...] = h
    out_ref[...] = jnp.dot(h, wc_ref[...], preferred_element_type=jnp.float32) + bc_ref[...]


def _acc_spec(lo):
    def imap(i):
        qn = i // (QN // BN)
        cc = jnp.clip(qn - lo, 0, 1)
        return (cc, i % (QN // BN), 0)
    return pl.BlockSpec((1, BN, H), imap)


_full2 = lambda r, c: pl.BlockSpec((r, c), lambda i: (0, 0))
_rows2 = lambda c: pl.BlockSpec((BN, c), lambda i: (i, 0))
_rows3 = lambda c: pl.BlockSpec((NC, BN, c), lambda i: (0, i, 0))

_k1_call = pl.pallas_call(
    _k1_body,
    grid=(N // BN,),
    in_specs=[_rows3(16), _rows2(D), _full2(D, H)],
    out_specs=[_rows2(1), _rows2(H)],
    out_shape=[jax.ShapeDtypeStruct((N, 1), jnp.float32),
               jax.ShapeDtypeStruct((N, H), jnp.float32)],
)

_k_mid_call = pl.pallas_call(
    _k_mid_body,
    grid=(N // BN,),
    in_specs=[_acc_spec(0), _acc_spec(2), _rows2(H), _rows2(1),
              _full2(1, H), _full2(H, H)],
    out_specs=_rows2(H),
    out_shape=jax.ShapeDtypeStruct((N, H), jnp.float32),
)

_k_last_call = pl.pallas_call(
    _k_last_body,
    grid=(N // BN,),
    in_specs=[_acc_spec(0), _acc_spec(2), _rows2(H), _rows2(1),
              _full2(1, H), _full2(H, C), _full2(1, C)],
    out_specs=[_rows2(C), _rows2(H)],
    out_shape=[jax.ShapeDtypeStruct((N, C), jnp.float32),
               jax.ShapeDtypeStruct((N, H), jnp.float32)],
)


def kernel(x, edge_index, W1, b1, W2, b2, W3, b3, Wc, bc):
    npad = E_PAD - E
    srcf = jnp.concatenate(
        [edge_index[0].astype(jnp.int32), jnp.zeros((npad,), jnp.int32)])
    dstf = jnp.concatenate(
        [edge_index[1].astype(jnp.int32),
         jnp.full((npad,), TRASH_DST, jnp.int32)])
    src2 = srcf.reshape(EROWS, K)
    dst2 = dstf.reshape(EROWS, K)

    _degree_kernel, _agg_kernels = _sc_kernels()
    dega = _degree_kernel(dst2)

    def layer_agg(y):
        accA = _agg_kernels[0](y, src2, dst2)
        accB = _agg_kernels[1](y, src2, dst2)
        return accA, accB

    dinv, y1 = _k1_call(dega, x, W1)
    accA, accB = layer_agg(y1)
    y2 = _k_mid_call(accA, accB, y1, dinv, b1.reshape(1, H), W2)
    accA, accB = layer_agg(y2)
    y3 = _k_mid_call(accA, accB, y2, dinv, b2.reshape(1, H), W3)
    accA, accB = layer_agg(y3)
    out, h = _k_last_call(accA, accB, y3, dinv, b3.reshape(1, H), Wc,
                          bc.reshape(1, C))
    return (out, h)
